# baseline (speedup 1.0000x reference)
import jax
import jax.numpy as jnp
from jax import lax
from jax.experimental import pallas as pl
from jax.experimental.pallas import tpu as pltpu
from jax.experimental.pallas import tpu_sc as plsc

B, C, N, M = 16, 256, 16384, 4096
NC, NS, L = 2, 16, 16
NW = NC * NS
ROWS_PER_W = (B * C) // NW
C_PER_W = C // (NW // B)
NBUF = 4


def _gather_body(features_hbm, indices_hbm, out_hbm, idx_v, row_v, res_v,
                 *sems):
    wid = lax.axis_index("s") * NC + lax.axis_index("c")
    b = wid // (NW // B)
    c0 = (wid % (NW // B)) * C_PER_W
    sins = sems[:NBUF]
    souts = sems[NBUF:]

    pltpu.sync_copy(indices_hbm.at[b], idx_v)

    def in_copy(r, buf):
        return pltpu.make_async_copy(
            features_hbm.at[b, c0 + r], row_v.at[pl.ds(buf * N, N)],
            sins[buf])

    def out_copy(r, buf):
        return pltpu.make_async_copy(
            res_v.at[pl.ds(buf * M, M)], out_hbm.at[b, c0 + r], souts[buf])

    for buf in range(NBUF):
        in_copy(buf, buf).start()
        out_copy(buf, buf).start()

    def ring_step(i, carry):
        r0 = i * NBUF
        for buf in range(NBUF):
            r = r0 + buf
            in_copy(r, buf).wait()
            out_copy(r, buf).wait()

            @plsc.parallel_loop(0, M // L, unroll=16)
            def _gather(j):
                iv = idx_v[pl.ds(j * L, L)] + (buf * N)
                res_v[pl.ds(buf * M + j * L, L)] = plsc.load_gather(
                    row_v, [iv])

            out_copy(r, buf).start()
            rn = jnp.minimum(r + NBUF, ROWS_PER_W - 1)
            in_copy(rn, buf).start()
        return carry

    lax.fori_loop(0, ROWS_PER_W // NBUF, ring_step, 0)

    for buf in range(NBUF):
        in_copy(0, buf).wait()
        out_copy(0, buf).wait()


@jax.jit
def kernel(features, indices):
    idx32 = indices.astype(jnp.int32)
    mesh = plsc.VectorSubcoreMesh(core_axis_name="c", subcore_axis_name="s")
    run = pl.kernel(
        _gather_body,
        out_type=jax.ShapeDtypeStruct((B, C, M), jnp.float32),
        mesh=mesh,
        scratch_types=(
            [pltpu.VMEM((M,), jnp.int32),
             pltpu.VMEM((NBUF * N,), jnp.float32),
             pltpu.VMEM((NBUF * M,), jnp.float32)]
            + [pltpu.SemaphoreType.DMA] * (2 * NBUF)
        ),
        compiler_params=pltpu.CompilerParams(needs_layout_passes=False),
    )
    return run(features, idx32)


# speedup vs baseline: 1.0211x; 1.0211x over previous
import jax
import jax.numpy as jnp
from jax import lax
from jax.experimental import pallas as pl
from jax.experimental.pallas import tpu as pltpu
from jax.experimental.pallas import tpu_sc as plsc

B, C, N, M = 16, 256, 16384, 4096
NC, NS, L = 2, 16, 16
NW = NC * NS
ROWS_PER_W = (B * C) // NW
C_PER_W = C // (NW // B)
NBUF = 4


def _gather_body(features_hbm, indices_hbm, out_hbm, idx_v, row_v, res_v,
                 *sems):
    wid = lax.axis_index("s") * NC + lax.axis_index("c")
    b = wid // (NW // B)
    c0 = (wid % (NW // B)) * C_PER_W
    sins = sems[:NBUF]
    souts = sems[NBUF:]

    pltpu.sync_copy(indices_hbm.at[b], idx_v)

    def in_copy(r, buf):
        return pltpu.make_async_copy(
            features_hbm.at[b, c0 + r], row_v.at[pl.ds(buf * N, N)],
            sins[buf])

    def out_copy(r, buf):
        return pltpu.make_async_copy(
            res_v.at[pl.ds(buf * M, M)], out_hbm.at[b, c0 + r], souts[buf])

    for buf in range(NBUF):
        in_copy(buf, buf).start()
        out_copy(buf, buf).start()

    def ring_step(i, carry):
        r0 = i * NBUF
        for buf in range(NBUF):
            r = r0 + buf
            in_copy(r, buf).wait()
            out_copy(r, buf).wait()

            @plsc.parallel_loop(0, M // L, unroll=8)
            def _gather(j):
                iv = idx_v[pl.ds(j * L, L)] + (buf * N)
                res_v[pl.ds(buf * M + j * L, L)] = plsc.load_gather(
                    row_v, [iv])

            out_copy(r, buf).start()

            @pl.when(r + NBUF < ROWS_PER_W)
            def _refill():
                in_copy(r + NBUF, buf).start()
        return carry

    lax.fori_loop(0, ROWS_PER_W // NBUF, ring_step, 0)

    for buf in range(NBUF):
        out_copy(0, buf).wait()


@jax.jit
def kernel(features, indices):
    idx32 = indices.astype(jnp.int32)
    mesh = plsc.VectorSubcoreMesh(core_axis_name="c", subcore_axis_name="s")
    run = pl.kernel(
        _gather_body,
        out_type=jax.ShapeDtypeStruct((B, C, M), jnp.float32),
        mesh=mesh,
        scratch_types=(
            [pltpu.VMEM((M,), jnp.int32),
             pltpu.VMEM((NBUF * N,), jnp.float32),
             pltpu.VMEM((NBUF * M,), jnp.float32)]
            + [pltpu.SemaphoreType.DMA] * (2 * NBUF)
        ),
        compiler_params=pltpu.CompilerParams(needs_layout_passes=False),
    )
    return run(features, idx32)
